# Initial kernel scaffold; baseline (speedup 1.0000x reference)
import jax
import jax.numpy as jnp
from jax.experimental import pallas as pl


def kernel(x_p, x_d, edge_attr_p, edge_attr_d, x_p_batch, edge_index_p, params):
    raise NotImplementedError("write your pallas kernel here")


# speedup vs baseline: 1.5923x; 1.5923x over previous
import functools

import jax
import jax.numpy as jnp
from jax import lax
from jax.experimental import pallas as pl
from jax.experimental.pallas import tpu as pltpu
from jax.experimental.pallas import tpu_sc as plsc

N = 10000
E = 160000
D_FEAT = 128
D_EDGE = 16
HID = 16
N_GRAPHS = 64

NC = 2
NS = 16
C = 128
NCHUNK = 40
EPT = C * NCHUNK
E_PAD = NC * NS * EPT
TW = HID * HID + HID
N_AGG = 10112
ROWS_PT = N_AGG // NS


def _h_body(ea_ref, w_ref, b_ref, h1_ref, h2_ref, h3_ref):
    h = jnp.maximum(jnp.dot(ea_ref[...], w_ref[...]) + b_ref[...], 0.0)
    h1_ref[...] = h[:, 0:HID]
    h2_ref[...] = h[:, HID:2 * HID]
    h3_ref[...] = h[:, 2 * HID:3 * HID]


def _edge_mlp(ea_pad, w1cat, b1cat):
    blk = 2048
    grid = E_PAD // blk
    return pl.pallas_call(
        _h_body,
        grid=(grid,),
        in_specs=[
            pl.BlockSpec((blk, D_EDGE), lambda i: (i, 0)),
            pl.BlockSpec((D_EDGE, 3 * HID), lambda i: (0, 0)),
            pl.BlockSpec((1, 3 * HID), lambda i: (0, 0)),
        ],
        out_specs=[
            pl.BlockSpec((blk, HID), lambda i: (i, 0)),
            pl.BlockSpec((blk, HID), lambda i: (i, 0)),
            pl.BlockSpec((blk, HID), lambda i: (i, 0)),
        ],
        out_shape=[jax.ShapeDtypeStruct((E_PAD, HID), jnp.float32)] * 3,
    )(ea_pad, w1cat, b1cat)


def _t1_body(x_ref, a_ref, t_ref, r_ref):
    t = jnp.dot(x_ref[...], a_ref[...])
    t_ref[...] = t[:, :TW]
    r_ref[...] = t[:, TW:TW + HID]


def _table_first(x, a):
    blk = 2000
    return pl.pallas_call(
        _t1_body,
        grid=(N // blk,),
        in_specs=[
            pl.BlockSpec((blk, D_FEAT), lambda i: (i, 0)),
            pl.BlockSpec((D_FEAT, TW + HID), lambda i: (0, 0)),
        ],
        out_specs=[
            pl.BlockSpec((blk, TW), lambda i: (i, 0)),
            pl.BlockSpec((blk, HID), lambda i: (i, 0)),
        ],
        out_shape=[
            jax.ShapeDtypeStruct((N, TW), jnp.float32),
            jax.ShapeDtypeStruct((N, HID), jnp.float32),
        ],
    )(x, a)


def _tn_body(a0_ref, a1_ref, r_ref, b_ref, a_ref, t_ref, rout_ref):
    x = jnp.maximum(a0_ref[...] + a1_ref[...] + r_ref[...] + b_ref[...], 0.0)
    t = jnp.dot(x, a_ref[...])
    t_ref[...] = t[:, :TW]
    rout_ref[...] = t[:, TW:TW + HID]


def _table_next(agg0, agg1, root, bias, a):
    blk = 2000
    return pl.pallas_call(
        _tn_body,
        grid=(N // blk,),
        in_specs=[
            pl.BlockSpec((blk, HID), lambda i: (i, 0)),
            pl.BlockSpec((blk, HID), lambda i: (i, 0)),
            pl.BlockSpec((blk, HID), lambda i: (i, 0)),
            pl.BlockSpec((1, HID), lambda i: (0, 0)),
            pl.BlockSpec((HID, TW + HID), lambda i: (0, 0)),
        ],
        out_specs=[
            pl.BlockSpec((blk, TW), lambda i: (i, 0)),
            pl.BlockSpec((blk, HID), lambda i: (i, 0)),
        ],
        out_shape=[
            jax.ShapeDtypeStruct((N, TW), jnp.float32),
            jax.ShapeDtypeStruct((N, HID), jnp.float32),
        ],
    )(agg0, agg1, root, bias, a)


def _pool_body(a0_ref, a1_ref, r_ref, bias_ref, batch_ref,
               w1_ref, b1_ref, w2_ref, b2_ref, out_ref):
    x = a0_ref[...] + a1_ref[...] + r_ref[...] + bias_ref[...]
    gid = lax.broadcasted_iota(jnp.int32, (N_GRAPHS, N), 0)
    p = (gid == batch_ref[...]).astype(jnp.float32)
    sums = jnp.dot(p, x)
    counts = jnp.sum(p, axis=1, keepdims=True)
    g = sums / jnp.maximum(counts, 1.0)
    g = jnp.maximum(jnp.dot(g, w1_ref[...]) + b1_ref[...], 0.0)
    out_ref[...] = jnp.dot(g, w2_ref[...]) + b2_ref[...]


def _pool_mlp(agg0, agg1, root, bias, batch_row, w1, b1, w2, b2):
    return pl.pallas_call(
        _pool_body,
        grid=(1,),
        in_specs=[
            pl.BlockSpec((N, HID), lambda i: (0, 0)),
            pl.BlockSpec((N, HID), lambda i: (0, 0)),
            pl.BlockSpec((N, HID), lambda i: (0, 0)),
            pl.BlockSpec((1, HID), lambda i: (0, 0)),
            pl.BlockSpec((1, N), lambda i: (0, 0)),
            pl.BlockSpec((HID, HID), lambda i: (0, 0)),
            pl.BlockSpec((1, HID), lambda i: (0, 0)),
            pl.BlockSpec((HID, 1), lambda i: (0, 0)),
            pl.BlockSpec((1, 1), lambda i: (0, 0)),
        ],
        out_specs=pl.BlockSpec((N_GRAPHS, 1), lambda i: (0, 0)),
        out_shape=jax.ShapeDtypeStruct((N_GRAPHS, 1), jnp.float32),
    )(agg0, agg1, root, bias, batch_row, w1, b1, w2, b2)


def _sc_edge_body(t_hbm, h_hbm, src_hbm, dst_hbm, zero_hbm, out_hbm,
                  src_v, dst_v, h_v, rows_v, msg_v, agg_sh, sem):
    c = lax.axis_index("c")
    s = lax.axis_index("s")
    wid = c * NS + s
    base0 = wid * EPT
    rs = s * ROWS_PT

    pltpu.sync_copy(zero_hbm, agg_sh.at[pl.ds(rs, ROWS_PT)])
    plsc.subcore_barrier()

    def chunk_body(ch, carry):
        base = base0 + ch * C
        pltpu.sync_copy(src_hbm.at[pl.ds(base, C)], src_v)
        pltpu.sync_copy(dst_hbm.at[pl.ds(base, C)], dst_v)
        pltpu.sync_copy(h_hbm.at[pl.ds(base, C)], h_v)
        pltpu.async_copy(t_hbm.at[src_v], rows_v, sem).wait()

        def edge_body(e, carry2):
            hvec = h_v[e]
            acc = rows_v[e, pl.ds(HID * HID, HID)]
            for k in range(HID):
                w = jnp.broadcast_to(hvec[k], (HID,))
                acc = acc + w * rows_v[e, pl.ds(k * HID, HID)]
            msg_v[e] = acc
            return carry2

        lax.fori_loop(0, C, edge_body, 0)
        pltpu.sync_copy(msg_v, agg_sh.at[dst_v], add=True)
        return carry

    lax.fori_loop(0, NCHUNK, chunk_body, 0)
    plsc.subcore_barrier()
    pltpu.sync_copy(agg_sh.at[pl.ds(rs, ROWS_PT)],
                    out_hbm.at[c, pl.ds(rs, ROWS_PT)])


@functools.lru_cache(maxsize=1)
def _build_sc_edge():
    mesh = plsc.VectorSubcoreMesh(
        core_axis_name="c", subcore_axis_name="s",
        num_cores=NC, num_subcores=NS)
    return pl.kernel(
        _sc_edge_body,
        out_type=jax.ShapeDtypeStruct((NC, N_AGG, HID), jnp.float32),
        mesh=mesh,
        compiler_params=pltpu.CompilerParams(use_tc_tiling_on_sc=False),
        scratch_types=[
            pltpu.VMEM((C,), jnp.int32),
            pltpu.VMEM((C,), jnp.int32),
            pltpu.VMEM((C, HID), jnp.float32),
            pltpu.VMEM((C, TW), jnp.float32),
            pltpu.VMEM((C, HID), jnp.float32),
            pltpu.VMEM_SHARED((N_AGG, HID), jnp.float32),
            pltpu.SemaphoreType.DMA,
        ],
    )


def _sc_edge(t, h, src_p, dst_p, zero_blk):
    return _build_sc_edge()(t, h, src_p, dst_p, zero_blk)


def _make_a(conv, in_dim):
    w2r = conv["nnW2"].reshape(HID, in_dim, HID)
    return jnp.concatenate(
        [
            w2r.transpose(1, 0, 2).reshape(in_dim, HID * HID),
            conv["nnb2"].reshape(in_dim, HID),
            conv["rootW"],
        ],
        axis=1,
    )


def kernel(x_p, x_d, edge_attr_p, edge_attr_d, x_p_batch, edge_index_p, params):
    src = edge_index_p[0]
    dst = edge_index_p[1]
    npad = E_PAD - E
    src_p = jnp.concatenate([src, jnp.zeros((npad,), jnp.int32)])
    dst_p = jnp.concatenate([dst, jnp.full((npad,), N, jnp.int32)])
    ea_pad = jnp.concatenate(
        [edge_attr_p, jnp.zeros((npad, D_EDGE), jnp.float32)], axis=0)

    convs = params["convs"]
    w1cat = jnp.concatenate([cv["nnW1"] for cv in convs], axis=1)
    b1cat = jnp.concatenate([cv["nnb1"] for cv in convs]).reshape(1, 3 * HID)
    a_mats = [
        _make_a(convs[0], D_FEAT),
        _make_a(convs[1], HID),
        _make_a(convs[2], HID),
    ]
    zero_blk = jnp.zeros((ROWS_PT, HID), jnp.float32)

    h1, h2, h3 = _edge_mlp(ea_pad, w1cat, b1cat)
    hs = [h1, h2, h3]

    t, root = _table_first(x_p, a_mats[0])
    for l in range(3):
        agg = _sc_edge(t, hs[l], src_p, dst_p, zero_blk)
        if l < 2:
            t, root = _table_next(
                agg[0], agg[1], root,
                convs[l]["bias"].reshape(1, HID), a_mats[l + 1])

    w1, b1 = params["lin1"]
    w2, b2 = params["lin2"]
    return _pool_mlp(
        agg[0], agg[1], root, convs[2]["bias"].reshape(1, HID),
        x_p_batch.reshape(1, N).astype(jnp.int32),
        w1, b1.reshape(1, HID), w2, b2.reshape(1, 1))


# speedup vs baseline: 2.1393x; 1.3436x over previous
import functools

import jax
import jax.numpy as jnp
from jax import lax
from jax.experimental import pallas as pl
from jax.experimental.pallas import tpu as pltpu
from jax.experimental.pallas import tpu_sc as plsc

N = 10000
E = 160000
D_FEAT = 128
D_EDGE = 16
HID = 16
N_GRAPHS = 64

NC = 2
NS = 16
C = 128
NCHUNK = 40
EPT = C * NCHUNK
E_PAD = NC * NS * EPT
TW = HID * HID + HID
N_AGG = 10112
ROWS_PT = N_AGG // NS


def _h_body(ea_ref, w_ref, b_ref, h1_ref, h2_ref, h3_ref):
    h = jnp.maximum(jnp.dot(ea_ref[...], w_ref[...]) + b_ref[...], 0.0)
    h1_ref[...] = h[:, 0:HID]
    h2_ref[...] = h[:, HID:2 * HID]
    h3_ref[...] = h[:, 2 * HID:3 * HID]


def _edge_mlp(ea_pad, w1cat, b1cat):
    blk = 2048
    grid = E_PAD // blk
    return pl.pallas_call(
        _h_body,
        grid=(grid,),
        in_specs=[
            pl.BlockSpec((blk, D_EDGE), lambda i: (i, 0)),
            pl.BlockSpec((D_EDGE, 3 * HID), lambda i: (0, 0)),
            pl.BlockSpec((1, 3 * HID), lambda i: (0, 0)),
        ],
        out_specs=[
            pl.BlockSpec((blk, HID), lambda i: (i, 0)),
            pl.BlockSpec((blk, HID), lambda i: (i, 0)),
            pl.BlockSpec((blk, HID), lambda i: (i, 0)),
        ],
        out_shape=[jax.ShapeDtypeStruct((E_PAD, HID), jnp.float32)] * 3,
    )(ea_pad, w1cat, b1cat)


def _t1_body(x_ref, a_ref, t_ref, r_ref):
    t = jnp.dot(x_ref[...], a_ref[...])
    t_ref[...] = t[:, :TW]
    r_ref[...] = t[:, TW:TW + HID]


def _table_first(x, a):
    blk = 2000
    return pl.pallas_call(
        _t1_body,
        grid=(N // blk,),
        in_specs=[
            pl.BlockSpec((blk, D_FEAT), lambda i: (i, 0)),
            pl.BlockSpec((D_FEAT, TW + HID), lambda i: (0, 0)),
        ],
        out_specs=[
            pl.BlockSpec((blk, TW), lambda i: (i, 0)),
            pl.BlockSpec((blk, HID), lambda i: (i, 0)),
        ],
        out_shape=[
            jax.ShapeDtypeStruct((N, TW), jnp.float32),
            jax.ShapeDtypeStruct((N, HID), jnp.float32),
        ],
    )(x, a)


def _tn_body(a0_ref, a1_ref, r_ref, b_ref, a_ref, t_ref, rout_ref):
    x = jnp.maximum(a0_ref[...] + a1_ref[...] + r_ref[...] + b_ref[...], 0.0)
    t = jnp.dot(x, a_ref[...])
    t_ref[...] = t[:, :TW]
    rout_ref[...] = t[:, TW:TW + HID]


def _table_next(agg0, agg1, root, bias, a):
    blk = 2000
    return pl.pallas_call(
        _tn_body,
        grid=(N // blk,),
        in_specs=[
            pl.BlockSpec((blk, HID), lambda i: (i, 0)),
            pl.BlockSpec((blk, HID), lambda i: (i, 0)),
            pl.BlockSpec((blk, HID), lambda i: (i, 0)),
            pl.BlockSpec((1, HID), lambda i: (0, 0)),
            pl.BlockSpec((HID, TW + HID), lambda i: (0, 0)),
        ],
        out_specs=[
            pl.BlockSpec((blk, TW), lambda i: (i, 0)),
            pl.BlockSpec((blk, HID), lambda i: (i, 0)),
        ],
        out_shape=[
            jax.ShapeDtypeStruct((N, TW), jnp.float32),
            jax.ShapeDtypeStruct((N, HID), jnp.float32),
        ],
    )(agg0, agg1, root, bias, a)


def _pool_body(a0_ref, a1_ref, r_ref, bias_ref, batch_ref,
               w1_ref, b1_ref, w2_ref, b2_ref, out_ref):
    x = a0_ref[...] + a1_ref[...] + r_ref[...] + bias_ref[...]
    gid = lax.broadcasted_iota(jnp.int32, (N_GRAPHS, N), 0)
    p = (gid == batch_ref[...]).astype(jnp.float32)
    sums = jnp.dot(p, x)
    counts = jnp.sum(p, axis=1, keepdims=True)
    g = sums / jnp.maximum(counts, 1.0)
    g = jnp.maximum(jnp.dot(g, w1_ref[...]) + b1_ref[...], 0.0)
    out_ref[...] = jnp.dot(g, w2_ref[...]) + b2_ref[...]


def _pool_mlp(agg0, agg1, root, bias, batch_row, w1, b1, w2, b2):
    return pl.pallas_call(
        _pool_body,
        grid=(1,),
        in_specs=[
            pl.BlockSpec((N, HID), lambda i: (0, 0)),
            pl.BlockSpec((N, HID), lambda i: (0, 0)),
            pl.BlockSpec((N, HID), lambda i: (0, 0)),
            pl.BlockSpec((1, HID), lambda i: (0, 0)),
            pl.BlockSpec((1, N), lambda i: (0, 0)),
            pl.BlockSpec((HID, HID), lambda i: (0, 0)),
            pl.BlockSpec((1, HID), lambda i: (0, 0)),
            pl.BlockSpec((HID, 1), lambda i: (0, 0)),
            pl.BlockSpec((1, 1), lambda i: (0, 0)),
        ],
        out_specs=pl.BlockSpec((N_GRAPHS, 1), lambda i: (0, 0)),
        out_shape=jax.ShapeDtypeStruct((N_GRAPHS, 1), jnp.float32),
    )(agg0, agg1, root, bias, batch_row, w1, b1, w2, b2)


def _sc_edge_body(t_hbm, h_hbm, ei_hbm, zero_hbm, out_hbm,
                  src_v0, src_v1, dst_v0, dst_v1, h_v0, h_v1,
                  rows_v0, rows_v1, msg_v0, msg_v1, agg_sh,
                  sem_i0, sem_i1, sem_d0, sem_d1, sem_h0, sem_h1,
                  sem_g0, sem_g1, sem_s0, sem_s1):
    src_v = (src_v0, src_v1)
    dst_v = (dst_v0, dst_v1)
    h_v = (h_v0, h_v1)
    rows_v = (rows_v0, rows_v1)
    msg_v = (msg_v0, msg_v1)
    sem_i = (sem_i0, sem_i1)
    sem_d = (sem_d0, sem_d1)
    sem_h = (sem_h0, sem_h1)
    sem_g = (sem_g0, sem_g1)
    sem_s = (sem_s0, sem_s1)

    c = lax.axis_index("c")
    s = lax.axis_index("s")
    wid = c * NS + s
    base0 = wid * EPT
    rs = s * ROWS_PT

    def issue_src(ch, b):
        pltpu.async_copy(ei_hbm.at[0, pl.ds(base0 + ch * C, C)],
                         src_v[b], sem_i[b])

    def issue_dsth(ch, b):
        pltpu.async_copy(ei_hbm.at[1, pl.ds(base0 + ch * C, C)],
                         dst_v[b], sem_d[b])
        pltpu.async_copy(h_hbm.at[pl.ds(base0 + ch * C, C)],
                         h_v[b], sem_h[b])

    pltpu.sync_copy(zero_hbm, agg_sh.at[pl.ds(rs, ROWS_PT)])
    plsc.subcore_barrier()

    issue_src(0, 0)
    pltpu.make_async_copy(ei_hbm.at[0, pl.ds(base0, C)], src_v[0],
                          sem_i[0]).wait()
    pltpu.async_copy(t_hbm.at[src_v[0]], rows_v[0], sem_g[0])
    issue_dsth(0, 0)
    issue_src(1, 1)

    def pair_body(i, carry):
        for b in (0, 1):
            ch = 2 * i + b
            o = 1 - b
            pltpu.make_async_copy(t_hbm.at[src_v[b]], rows_v[b],
                                  sem_g[b]).wait()

            @pl.when(ch + 2 < NCHUNK)
            def _():
                issue_src(ch + 2, b)

            @pl.when(ch + 1 < NCHUNK)
            def _():
                pltpu.make_async_copy(
                    ei_hbm.at[0, pl.ds(base0, C)], src_v[o], sem_i[o]).wait()
                pltpu.async_copy(t_hbm.at[src_v[o]], rows_v[o], sem_g[o])

                @pl.when(ch >= 1)
                def _():
                    pltpu.make_async_copy(
                        msg_v[o], agg_sh.at[dst_v[o]], sem_s[o]).wait()

                issue_dsth(ch + 1, o)

            pltpu.make_async_copy(
                h_hbm.at[pl.ds(base0, C)], h_v[b], sem_h[b]).wait()

            def edge_body(e, carry2):
                hvec = h_v[b][e]
                acc = rows_v[b][e, pl.ds(HID * HID, HID)]
                for k in range(HID):
                    w = jnp.broadcast_to(hvec[k], (HID,))
                    acc = acc + w * rows_v[b][e, pl.ds(k * HID, HID)]
                msg_v[b][e] = acc
                return carry2

            lax.fori_loop(0, C, edge_body, 0)
            pltpu.make_async_copy(
                ei_hbm.at[1, pl.ds(base0, C)], dst_v[b], sem_d[b]).wait()
            pltpu.async_copy(msg_v[b], agg_sh.at[dst_v[b]], sem_s[b],
                             add=True)
        return carry

    lax.fori_loop(0, NCHUNK // 2, pair_body, 0)
    for b in (0, 1):
        pltpu.make_async_copy(msg_v[b], agg_sh.at[dst_v[b]], sem_s[b]).wait()
    plsc.subcore_barrier()
    pltpu.sync_copy(agg_sh.at[pl.ds(rs, ROWS_PT)],
                    out_hbm.at[c, pl.ds(rs, ROWS_PT)])


@functools.lru_cache(maxsize=1)
def _build_sc_edge():
    mesh = plsc.VectorSubcoreMesh(
        core_axis_name="c", subcore_axis_name="s",
        num_cores=NC, num_subcores=NS)
    return pl.kernel(
        _sc_edge_body,
        out_type=jax.ShapeDtypeStruct((NC, N_AGG, HID), jnp.float32),
        mesh=mesh,
        compiler_params=pltpu.CompilerParams(use_tc_tiling_on_sc=False),
        scratch_types=(
            [pltpu.VMEM((C,), jnp.int32)] * 4 +
            [pltpu.VMEM((C, HID), jnp.float32)] * 2 +
            [pltpu.VMEM((C, TW), jnp.float32)] * 2 +
            [pltpu.VMEM((C, HID), jnp.float32)] * 2 +
            [pltpu.VMEM_SHARED((N_AGG, HID), jnp.float32)] +
            [pltpu.SemaphoreType.DMA] * 10
        ),
    )


def _sc_edge(t, h, ei_pad, zero_blk):
    return _build_sc_edge()(t, h, ei_pad, zero_blk)


def _make_a(conv, in_dim):
    w2r = conv["nnW2"].reshape(HID, in_dim, HID)
    return jnp.concatenate(
        [
            w2r.transpose(1, 0, 2).reshape(in_dim, HID * HID),
            conv["nnb2"].reshape(in_dim, HID),
            conv["rootW"],
        ],
        axis=1,
    )


def kernel(x_p, x_d, edge_attr_p, edge_attr_d, x_p_batch, edge_index_p, params):
    npad = E_PAD - E
    ei_pad = jnp.concatenate(
        [edge_index_p,
         jnp.stack([jnp.zeros((npad,), jnp.int32),
                    jnp.full((npad,), N, jnp.int32)])], axis=1)
    ea_pad = jnp.concatenate(
        [edge_attr_p, jnp.zeros((npad, D_EDGE), jnp.float32)], axis=0)

    convs = params["convs"]
    w1cat = jnp.concatenate([cv["nnW1"] for cv in convs], axis=1)
    b1cat = jnp.concatenate([cv["nnb1"] for cv in convs]).reshape(1, 3 * HID)
    a_mats = [
        _make_a(convs[0], D_FEAT),
        _make_a(convs[1], HID),
        _make_a(convs[2], HID),
    ]
    zero_blk = jnp.zeros((ROWS_PT, HID), jnp.float32)

    h1, h2, h3 = _edge_mlp(ea_pad, w1cat, b1cat)
    hs = [h1, h2, h3]

    t, root = _table_first(x_p, a_mats[0])
    for l in range(3):
        agg = _sc_edge(t, hs[l], ei_pad, zero_blk)
        if l < 2:
            t, root = _table_next(
                agg[0], agg[1], root,
                convs[l]["bias"].reshape(1, HID), a_mats[l + 1])

    w1, b1 = params["lin1"]
    w2, b2 = params["lin2"]
    return _pool_mlp(
        agg[0], agg[1], root, convs[2]["bias"].reshape(1, HID),
        x_p_batch.reshape(1, N).astype(jnp.int32),
        w1, b1.reshape(1, HID), w2, b2.reshape(1, 1))


# speedup vs baseline: 2.1446x; 1.0025x over previous
import functools

import jax
import jax.numpy as jnp
from jax import lax
from jax.experimental import pallas as pl
from jax.experimental.pallas import tpu as pltpu
from jax.experimental.pallas import tpu_sc as plsc

N = 10000
E = 160000
D_FEAT = 128
D_EDGE = 16
HID = 16
N_GRAPHS = 64

NC = 2
NS = 16
C = 128
NCHUNK = 40
EPT = C * NCHUNK
E_PAD = NC * NS * EPT
TW = HID * HID + HID
N_AGG = 10112
ROWS_PT = N_AGG // NS


def _h_body(ea_ref, w_ref, b_ref, h1_ref, h2_ref, h3_ref):
    h = jnp.maximum(jnp.dot(ea_ref[...], w_ref[...]) + b_ref[...], 0.0)
    h1_ref[...] = h[:, 0:HID]
    h2_ref[...] = h[:, HID:2 * HID]
    h3_ref[...] = h[:, 2 * HID:3 * HID]


def _edge_mlp(ea_pad, w1cat, b1cat):
    blk = 2048
    grid = E_PAD // blk
    return pl.pallas_call(
        _h_body,
        grid=(grid,),
        in_specs=[
            pl.BlockSpec((blk, D_EDGE), lambda i: (i, 0)),
            pl.BlockSpec((D_EDGE, 3 * HID), lambda i: (0, 0)),
            pl.BlockSpec((1, 3 * HID), lambda i: (0, 0)),
        ],
        out_specs=[
            pl.BlockSpec((blk, HID), lambda i: (i, 0)),
            pl.BlockSpec((blk, HID), lambda i: (i, 0)),
            pl.BlockSpec((blk, HID), lambda i: (i, 0)),
        ],
        out_shape=[jax.ShapeDtypeStruct((E_PAD, HID), jnp.float32)] * 3,
    )(ea_pad, w1cat, b1cat)


def _t1_body(x_ref, a_ref, t_ref, r_ref):
    t = jnp.dot(x_ref[...], a_ref[...])
    t_ref[...] = t[:, :TW]
    r_ref[...] = t[:, TW:TW + HID]


def _table_first(x, a):
    blk = 2000
    return pl.pallas_call(
        _t1_body,
        grid=(N // blk,),
        in_specs=[
            pl.BlockSpec((blk, D_FEAT), lambda i: (i, 0)),
            pl.BlockSpec((D_FEAT, TW + HID), lambda i: (0, 0)),
        ],
        out_specs=[
            pl.BlockSpec((blk, TW), lambda i: (i, 0)),
            pl.BlockSpec((blk, HID), lambda i: (i, 0)),
        ],
        out_shape=[
            jax.ShapeDtypeStruct((N, TW), jnp.float32),
            jax.ShapeDtypeStruct((N, HID), jnp.float32),
        ],
    )(x, a)


def _tn_body(a0_ref, a1_ref, r_ref, b_ref, a_ref, t_ref, rout_ref):
    x = jnp.maximum(a0_ref[...] + a1_ref[...] + r_ref[...] + b_ref[...], 0.0)
    t = jnp.dot(x, a_ref[...])
    t_ref[...] = t[:, :TW]
    rout_ref[...] = t[:, TW:TW + HID]


def _table_next(agg0, agg1, root, bias, a):
    blk = 2000
    return pl.pallas_call(
        _tn_body,
        grid=(N // blk,),
        in_specs=[
            pl.BlockSpec((blk, HID), lambda i: (i, 0)),
            pl.BlockSpec((blk, HID), lambda i: (i, 0)),
            pl.BlockSpec((blk, HID), lambda i: (i, 0)),
            pl.BlockSpec((1, HID), lambda i: (0, 0)),
            pl.BlockSpec((HID, TW + HID), lambda i: (0, 0)),
        ],
        out_specs=[
            pl.BlockSpec((blk, TW), lambda i: (i, 0)),
            pl.BlockSpec((blk, HID), lambda i: (i, 0)),
        ],
        out_shape=[
            jax.ShapeDtypeStruct((N, TW), jnp.float32),
            jax.ShapeDtypeStruct((N, HID), jnp.float32),
        ],
    )(agg0, agg1, root, bias, a)


def _pool_body(a0_ref, a1_ref, r_ref, bias_ref, batch_ref,
               w1_ref, b1_ref, w2_ref, b2_ref, out_ref):
    x = a0_ref[...] + a1_ref[...] + r_ref[...] + bias_ref[...]
    gid = lax.broadcasted_iota(jnp.int32, (N_GRAPHS, N), 0)
    p = (gid == batch_ref[...]).astype(jnp.float32)
    sums = jnp.dot(p, x)
    counts = jnp.sum(p, axis=1, keepdims=True)
    g = sums / jnp.maximum(counts, 1.0)
    g = jnp.maximum(jnp.dot(g, w1_ref[...]) + b1_ref[...], 0.0)
    out_ref[...] = jnp.dot(g, w2_ref[...]) + b2_ref[...]


def _pool_mlp(agg0, agg1, root, bias, batch_row, w1, b1, w2, b2):
    return pl.pallas_call(
        _pool_body,
        grid=(1,),
        in_specs=[
            pl.BlockSpec((N, HID), lambda i: (0, 0)),
            pl.BlockSpec((N, HID), lambda i: (0, 0)),
            pl.BlockSpec((N, HID), lambda i: (0, 0)),
            pl.BlockSpec((1, HID), lambda i: (0, 0)),
            pl.BlockSpec((1, N), lambda i: (0, 0)),
            pl.BlockSpec((HID, HID), lambda i: (0, 0)),
            pl.BlockSpec((1, HID), lambda i: (0, 0)),
            pl.BlockSpec((HID, 1), lambda i: (0, 0)),
            pl.BlockSpec((1, 1), lambda i: (0, 0)),
        ],
        out_specs=pl.BlockSpec((N_GRAPHS, 1), lambda i: (0, 0)),
        out_shape=jax.ShapeDtypeStruct((N_GRAPHS, 1), jnp.float32),
    )(agg0, agg1, root, bias, batch_row, w1, b1, w2, b2)


def _sc_edge_body(t_hbm, h_hbm, ei_hbm, zero_hbm, out_hbm,
                  src_v0, src_v1, dst_v0, dst_v1, h_v0, h_v1,
                  rows_v0, rows_v1, msg_v0, msg_v1, agg_sh,
                  sem_i0, sem_i1, sem_d0, sem_d1, sem_h0, sem_h1,
                  sem_g0, sem_g1, sem_s0, sem_s1):
    src_v = (src_v0, src_v1)
    dst_v = (dst_v0, dst_v1)
    h_v = (h_v0, h_v1)
    rows_v = (rows_v0, rows_v1)
    msg_v = (msg_v0, msg_v1)
    sem_i = (sem_i0, sem_i1)
    sem_d = (sem_d0, sem_d1)
    sem_h = (sem_h0, sem_h1)
    sem_g = (sem_g0, sem_g1)
    sem_s = (sem_s0, sem_s1)

    c = lax.axis_index("c")
    s = lax.axis_index("s")
    wid = c * NS + s
    base0 = wid * EPT
    rs = s * ROWS_PT

    def issue_src(ch, b):
        pltpu.async_copy(ei_hbm.at[0, pl.ds(base0 + ch * C, C)],
                         src_v[b], sem_i[b])

    def issue_dsth(ch, b):
        pltpu.async_copy(ei_hbm.at[1, pl.ds(base0 + ch * C, C)],
                         dst_v[b], sem_d[b])
        pltpu.async_copy(h_hbm.at[pl.ds(base0 + ch * C, C)],
                         h_v[b], sem_h[b])

    pltpu.sync_copy(zero_hbm, agg_sh.at[pl.ds(rs, ROWS_PT)])
    plsc.subcore_barrier()

    issue_src(0, 0)
    pltpu.make_async_copy(ei_hbm.at[0, pl.ds(base0, C)], src_v[0],
                          sem_i[0]).wait()
    pltpu.async_copy(t_hbm.at[src_v[0]], rows_v[0], sem_g[0])
    issue_dsth(0, 0)
    issue_src(1, 1)

    def pair_body(i, carry):
        for b in (0, 1):
            ch = 2 * i + b
            o = 1 - b
            pltpu.make_async_copy(t_hbm.at[src_v[b]], rows_v[b],
                                  sem_g[b]).wait()

            @pl.when(ch + 2 < NCHUNK)
            def _():
                issue_src(ch + 2, b)

            @pl.when(ch + 1 < NCHUNK)
            def _():
                pltpu.make_async_copy(
                    ei_hbm.at[0, pl.ds(base0, C)], src_v[o], sem_i[o]).wait()
                pltpu.async_copy(t_hbm.at[src_v[o]], rows_v[o], sem_g[o])

                @pl.when(ch >= 1)
                def _():
                    pltpu.make_async_copy(
                        msg_v[o], agg_sh.at[dst_v[o]], sem_s[o]).wait()

                issue_dsth(ch + 1, o)

            pltpu.make_async_copy(
                h_hbm.at[pl.ds(base0, C)], h_v[b], sem_h[b]).wait()

            @plsc.parallel_loop(0, C, unroll=4)
            def edge_body(e):
                hvec = h_v[b][e]
                acc = rows_v[b][e, pl.ds(HID * HID, HID)]
                for k in range(HID):
                    w = jnp.broadcast_to(hvec[k], (HID,))
                    acc = acc + w * rows_v[b][e, pl.ds(k * HID, HID)]
                msg_v[b][e] = acc
            pltpu.make_async_copy(
                ei_hbm.at[1, pl.ds(base0, C)], dst_v[b], sem_d[b]).wait()
            pltpu.async_copy(msg_v[b], agg_sh.at[dst_v[b]], sem_s[b],
                             add=True)
        return carry

    lax.fori_loop(0, NCHUNK // 2, pair_body, 0)
    for b in (0, 1):
        pltpu.make_async_copy(msg_v[b], agg_sh.at[dst_v[b]], sem_s[b]).wait()
    plsc.subcore_barrier()
    pltpu.sync_copy(agg_sh.at[pl.ds(rs, ROWS_PT)],
                    out_hbm.at[c, pl.ds(rs, ROWS_PT)])


@functools.lru_cache(maxsize=1)
def _build_sc_edge():
    mesh = plsc.VectorSubcoreMesh(
        core_axis_name="c", subcore_axis_name="s",
        num_cores=NC, num_subcores=NS)
    return pl.kernel(
        _sc_edge_body,
        out_type=jax.ShapeDtypeStruct((NC, N_AGG, HID), jnp.float32),
        mesh=mesh,
        compiler_params=pltpu.CompilerParams(use_tc_tiling_on_sc=False),
        scratch_types=(
            [pltpu.VMEM((C,), jnp.int32)] * 4 +
            [pltpu.VMEM((C, HID), jnp.float32)] * 2 +
            [pltpu.VMEM((C, TW), jnp.float32)] * 2 +
            [pltpu.VMEM((C, HID), jnp.float32)] * 2 +
            [pltpu.VMEM_SHARED((N_AGG, HID), jnp.float32)] +
            [pltpu.SemaphoreType.DMA] * 10
        ),
    )


def _sc_edge(t, h, ei_pad, zero_blk):
    return _build_sc_edge()(t, h, ei_pad, zero_blk)


def _make_a(conv, in_dim):
    w2r = conv["nnW2"].reshape(HID, in_dim, HID)
    return jnp.concatenate(
        [
            w2r.transpose(1, 0, 2).reshape(in_dim, HID * HID),
            conv["nnb2"].reshape(in_dim, HID),
            conv["rootW"],
        ],
        axis=1,
    )


def kernel(x_p, x_d, edge_attr_p, edge_attr_d, x_p_batch, edge_index_p, params):
    npad = E_PAD - E
    ei_pad = jnp.concatenate(
        [edge_index_p,
         jnp.stack([jnp.zeros((npad,), jnp.int32),
                    jnp.full((npad,), N, jnp.int32)])], axis=1)
    ea_pad = jnp.concatenate(
        [edge_attr_p, jnp.zeros((npad, D_EDGE), jnp.float32)], axis=0)

    convs = params["convs"]
    w1cat = jnp.concatenate([cv["nnW1"] for cv in convs], axis=1)
    b1cat = jnp.concatenate([cv["nnb1"] for cv in convs]).reshape(1, 3 * HID)
    a_mats = [
        _make_a(convs[0], D_FEAT),
        _make_a(convs[1], HID),
        _make_a(convs[2], HID),
    ]
    zero_blk = jnp.zeros((ROWS_PT, HID), jnp.float32)

    h1, h2, h3 = _edge_mlp(ea_pad, w1cat, b1cat)
    hs = [h1, h2, h3]

    t, root = _table_first(x_p, a_mats[0])
    for l in range(3):
        agg = _sc_edge(t, hs[l], ei_pad, zero_blk)
        if l < 2:
            t, root = _table_next(
                agg[0], agg[1], root,
                convs[l]["bias"].reshape(1, HID), a_mats[l + 1])

    w1, b1 = params["lin1"]
    w2, b2 = params["lin2"]
    return _pool_mlp(
        agg[0], agg[1], root, convs[2]["bias"].reshape(1, HID),
        x_p_batch.reshape(1, N).astype(jnp.int32),
        w1, b1.reshape(1, HID), w2, b2.reshape(1, 1))
